# Initial kernel scaffold; baseline (speedup 1.0000x reference)
import jax
import jax.numpy as jnp
from jax.experimental import pallas as pl


def kernel(z, codebook):
    raise NotImplementedError("write your pallas kernel here")


# speedup vs baseline: 4.3829x; 4.3829x over previous
import functools

import jax
import jax.numpy as jnp
from jax import lax
from jax.experimental import pallas as pl
from jax.experimental.pallas import tpu as pltpu
from jax.experimental.pallas import tpu_sc as plsc

CODEBOOK_SIZE = 512
CODE_DIM = 32
N_TOKENS = 16 * 1024
BT = 2048
NB = N_TOKENS // BT

_NC, _NS = 2, 16
_NW = _NC * _NS
_B_PER_W = N_TOKENS // _NW


def _argmin_body(z_ref, cbt_ref, out_ref):
    zb = z_ref[...]
    cbt = cbt_ref[...]
    cbn = jnp.sum(cbt * cbt, axis=0)
    dots = jnp.dot(
        zb, cbt,
        precision=lax.Precision.HIGHEST,
        preferred_element_type=jnp.float32,
    )
    scores = cbn[None, :] - 2.0 * dots
    idx = jnp.argmin(scores, axis=1).astype(jnp.int32)
    out_ref[0, 0, :] = idx


def _tc_argmin(zf, cbt):
    out = pl.pallas_call(
        _argmin_body,
        grid=(NB,),
        in_specs=[
            pl.BlockSpec((BT, CODE_DIM), lambda i: (i, 0)),
            pl.BlockSpec((CODE_DIM, CODEBOOK_SIZE), lambda i: (0, 0)),
        ],
        out_specs=pl.BlockSpec((1, 1, BT), lambda i: (i, 0, 0)),
        out_shape=jax.ShapeDtypeStruct((NB, 1, BT), jnp.int32),
    )(zf, cbt)
    return out.reshape(N_TOKENS)


_PAD_W = 128


@functools.partial(
    pl.kernel,
    mesh=plsc.VectorSubcoreMesh(core_axis_name="c", subcore_axis_name="s"),
    out_type=jax.ShapeDtypeStruct((N_TOKENS, _PAD_W), jnp.float32),
    scratch_types=[
        pltpu.VMEM((_B_PER_W,), jnp.int32),
        pltpu.VMEM((_B_PER_W, _PAD_W), jnp.float32),
        pltpu.SemaphoreType.DMA,
    ],
)
def _sc_gather(cb_hbm, idx_hbm, out_hbm, idx_v, rows_v, sem):
    wid = lax.axis_index("s") * _NC + lax.axis_index("c")
    base = wid * _B_PER_W
    pltpu.sync_copy(idx_hbm.at[pl.ds(base, _B_PER_W)], idx_v)
    pltpu.async_copy(cb_hbm.at[idx_v], rows_v, sem).wait()
    pltpu.sync_copy(rows_v, out_hbm.at[pl.ds(base, _B_PER_W)])


def kernel(z, codebook):
    zf = z.reshape(N_TOKENS, CODE_DIM)
    idx = _tc_argmin(zf, codebook.T)
    cb_pad = jnp.pad(codebook, ((0, 0), (0, _PAD_W - CODE_DIM)))
    zq = _sc_gather(cb_pad, idx)[:, :CODE_DIM]
    return zq.reshape(z.shape)


# speedup vs baseline: 5.8834x; 1.3424x over previous
import functools

import jax
import jax.numpy as jnp
from jax import lax
from jax.experimental import pallas as pl
from jax.experimental.pallas import tpu as pltpu
from jax.experimental.pallas import tpu_sc as plsc

CODEBOOK_SIZE = 512
CODE_DIM = 32
N_TOKENS = 16 * 1024
BT = 2048
NB = N_TOKENS // BT

_NC, _NS = 2, 16
_NW = _NC * _NS
_B_PER_W = N_TOKENS // _NW


def _split3(x):
    hi = x.astype(jnp.bfloat16)
    r = x - hi.astype(jnp.float32)
    mid = r.astype(jnp.bfloat16)
    lo = (r - mid.astype(jnp.float32)).astype(jnp.bfloat16)
    return hi, mid, lo


def _argmin_body(z_ref, cbt_ref, out_ref):
    zb = z_ref[...]
    cbt = cbt_ref[...]
    cbn = jnp.sum(cbt * cbt, axis=0)
    z_hi, z_mid, z_lo = _split3(zb)
    c_hi, c_mid, c_lo = _split3(cbt)
    z6 = jnp.concatenate([z_hi, z_hi, z_mid, z_mid, z_hi, z_lo], axis=1)
    c6 = jnp.concatenate([c_hi, c_mid, c_hi, c_mid, c_lo, c_hi], axis=0)
    dots = jnp.dot(z6, c6, preferred_element_type=jnp.float32)
    scores = cbn[None, :] - 2.0 * dots
    idx = jnp.argmin(scores, axis=1).astype(jnp.int32)
    out_ref[0, 0, :] = idx


def _tc_argmin(zf, cbt):
    out = pl.pallas_call(
        _argmin_body,
        grid=(NB,),
        in_specs=[
            pl.BlockSpec((BT, CODE_DIM), lambda i: (i, 0)),
            pl.BlockSpec((CODE_DIM, CODEBOOK_SIZE), lambda i: (0, 0)),
        ],
        out_specs=pl.BlockSpec((1, 1, BT), lambda i: (i, 0, 0)),
        out_shape=jax.ShapeDtypeStruct((NB, 1, BT), jnp.int32),
    )(zf, cbt)
    return out.reshape(N_TOKENS)


_PAD_W = 128
_LANES = 16


@functools.partial(
    pl.kernel,
    mesh=plsc.VectorSubcoreMesh(core_axis_name="c", subcore_axis_name="s"),
    out_type=jax.ShapeDtypeStruct((N_TOKENS * CODE_DIM,), jnp.float32),
    scratch_types=[
        pltpu.VMEM((_B_PER_W,), jnp.int32),
        pltpu.VMEM((_B_PER_W, _PAD_W), jnp.float32),
        pltpu.VMEM((_B_PER_W * CODE_DIM,), jnp.float32),
        pltpu.SemaphoreType.DMA,
    ],
)
def _sc_gather(cb_hbm, idx_hbm, out_hbm, idx_v, rows_v, comp_v, sem):
    wid = lax.axis_index("s") * _NC + lax.axis_index("c")
    base = wid * _B_PER_W
    pltpu.sync_copy(idx_hbm.at[pl.ds(base, _B_PER_W)], idx_v)
    pltpu.async_copy(cb_hbm.at[idx_v], rows_v, sem).wait()

    @pl.loop(0, _B_PER_W)
    def _(r):
        comp_v.at[pl.ds(r * CODE_DIM, _LANES)][...] = (
            rows_v.at[r, pl.ds(0, _LANES)][...])
        comp_v.at[pl.ds(r * CODE_DIM + _LANES, _LANES)][...] = (
            rows_v.at[r, pl.ds(_LANES, _LANES)][...])

    pltpu.sync_copy(
        comp_v, out_hbm.at[pl.ds(base * CODE_DIM, _B_PER_W * CODE_DIM)])


def kernel(z, codebook):
    zf = z.reshape(N_TOKENS, CODE_DIM)
    idx = _tc_argmin(zf, codebook.T)
    cb_pad = jnp.pad(codebook, ((0, 0), (0, _PAD_W - CODE_DIM)))
    zq = _sc_gather(cb_pad, idx)
    return zq.reshape(z.shape)


# speedup vs baseline: 6.6450x; 1.1295x over previous
import functools

import jax
import jax.numpy as jnp
from jax import lax
from jax.experimental import pallas as pl
from jax.experimental.pallas import tpu as pltpu
from jax.experimental.pallas import tpu_sc as plsc

CODEBOOK_SIZE = 512
CODE_DIM = 32
N_TOKENS = 16 * 1024
BT = 2048
NB = N_TOKENS // BT

_NC, _NS = 2, 16
_NW = _NC * _NS
_B_PER_W = N_TOKENS // _NW

_PAD_W = 128
_LANES = 16


def _split3(x):
    hi = x.astype(jnp.bfloat16)
    r = x - hi.astype(jnp.float32)
    mid = r.astype(jnp.bfloat16)
    lo = (r - mid.astype(jnp.float32)).astype(jnp.bfloat16)
    return hi, mid, lo


def _argmin_body(z_ref, cb_ref, out_ref, cbp_ref):
    zb = z_ref[...]
    cb = cb_ref[...]
    cbp_ref[...] = jnp.concatenate(
        [cb, jnp.zeros((CODEBOOK_SIZE, _PAD_W - CODE_DIM), jnp.float32)],
        axis=1)
    cbn = jnp.sum(cb * cb, axis=1)
    z_hi, z_mid, z_lo = _split3(zb.T)
    c_hi, c_mid, c_lo = _split3(-2.0 * cb)
    z6 = jnp.concatenate([z_hi, z_mid, z_hi, z_mid, z_lo, z_hi], axis=0)
    c6 = jnp.concatenate([c_hi, c_hi, c_mid, c_mid, c_hi, c_lo], axis=1)
    dots = jnp.dot(c6, z6, preferred_element_type=jnp.float32)
    scores = cbn[:, None] + dots
    idx = jnp.argmin(scores, axis=0).astype(jnp.int32)
    out_ref[0, 0, :] = idx


def _tc_argmin(zf, cb):
    idx, cb_pad = pl.pallas_call(
        _argmin_body,
        grid=(NB,),
        in_specs=[
            pl.BlockSpec((BT, CODE_DIM), lambda i: (i, 0)),
            pl.BlockSpec((CODEBOOK_SIZE, CODE_DIM), lambda i: (0, 0)),
        ],
        out_specs=[
            pl.BlockSpec((1, 1, BT), lambda i: (i, 0, 0)),
            pl.BlockSpec((CODEBOOK_SIZE, _PAD_W), lambda i: (0, 0)),
        ],
        out_shape=[
            jax.ShapeDtypeStruct((NB, 1, BT), jnp.int32),
            jax.ShapeDtypeStruct((CODEBOOK_SIZE, _PAD_W), jnp.float32),
        ],
    )(zf, cb)
    return idx.reshape(N_TOKENS), cb_pad


@functools.partial(
    pl.kernel,
    mesh=plsc.VectorSubcoreMesh(core_axis_name="c", subcore_axis_name="s"),
    out_type=jax.ShapeDtypeStruct((N_TOKENS * CODE_DIM,), jnp.float32),
    scratch_types=[
        pltpu.VMEM((_B_PER_W,), jnp.int32),
        pltpu.VMEM((_B_PER_W, _PAD_W), jnp.float32),
        pltpu.VMEM((_B_PER_W * CODE_DIM,), jnp.float32),
        pltpu.SemaphoreType.DMA,
    ],
)
def _sc_gather(cb_hbm, idx_hbm, out_hbm, idx_v, rows_v, comp_v, sem):
    wid = lax.axis_index("s") * _NC + lax.axis_index("c")
    base = wid * _B_PER_W
    pltpu.sync_copy(idx_hbm.at[pl.ds(base, _B_PER_W)], idx_v)
    pltpu.async_copy(cb_hbm.at[idx_v], rows_v, sem).wait()

    @pl.loop(0, _B_PER_W, step=4)
    def _(r0):
        for u in range(4):
            r = r0 + u
            comp_v.at[pl.ds(r * CODE_DIM, _LANES)][...] = (
                rows_v.at[r, pl.ds(0, _LANES)][...])
            comp_v.at[pl.ds(r * CODE_DIM + _LANES, _LANES)][...] = (
                rows_v.at[r, pl.ds(_LANES, _LANES)][...])

    pltpu.sync_copy(
        comp_v, out_hbm.at[pl.ds(base * CODE_DIM, _B_PER_W * CODE_DIM)])


def kernel(z, codebook):
    zf = z.reshape(N_TOKENS, CODE_DIM)
    idx, cb_pad = _tc_argmin(zf, codebook)
    zq = _sc_gather(cb_pad, idx)
    return zq.reshape(z.shape)
